# baseline (speedup 1.0000x reference)
import functools

import jax
import jax.numpy as jnp
from jax import lax
from jax.experimental import pallas as pl
from jax.experimental.pallas import tpu as pltpu
from jax.experimental.pallas import tpu_sc as plsc

N, S, A = 1024, 200, 3
E = 128
NROWS = N * S
NC, NSUB = 2, 16
NW = NC * NSUB
ROWS_PER_W = NROWS // NW
GSUB = 128
KSUB = 2
G = GSUB * KSUB
NG = ROWS_PER_W // G


def _sc_body(t0, t1, t2, idx_hbm, out_hbm, idxv, buf, sem0, sem1):
    c = lax.axis_index("c")
    s = lax.axis_index("s")
    wid = s * NC + c
    pltpu.sync_copy(idx_hbm.at[wid], idxv)
    tabs = (t0, t1, t2)
    sems = (sem0, sem1)

    def zero(p):
        z = jnp.zeros((16,), jnp.float32)

        def row(r, carry):
            for cc in range(E // 16):
                buf[p, r, pl.ds(cc * 16, 16)] = z
            return carry

        lax.fori_loop(0, G, row, 0)

    def issue(g, p):
        for a in range(A):
            for k in range(KSUB):
                pltpu.async_copy(
                    tabs[a].at[idxv.at[a, g, k]],
                    buf.at[p, pl.ds(k * GSUB, GSUB)],
                    sems[p],
                    add=True,
                )

    def wait(g, p):
        for a in range(A):
            for k in range(KSUB):
                pltpu.make_async_copy(
                    tabs[a].at[idxv.at[a, g, k]],
                    buf.at[p, pl.ds(k * GSUB, GSUB)],
                    sems[p],
                ).wait()

    def out(g, p):
        base = (wid * NG + g) * G
        pltpu.sync_copy(buf.at[p], out_hbm.at[pl.ds(base, G)])

    zero(0)
    zero(1)
    issue(0, 0)

    def pair(i, carry):
        g = 2 * i
        issue(g + 1, 1)
        wait(g, 0)
        out(g, 0)
        zero(0)
        issue(g + 2, 0)
        wait(g + 1, 1)
        out(g + 1, 1)
        zero(1)
        return carry

    lax.fori_loop(0, NG // 2, pair, 0)
    wait(NG - 1, 0)
    out(NG - 1, 0)


_mesh = plsc.VectorSubcoreMesh(
    core_axis_name="c", subcore_axis_name="s", num_cores=NC, num_subcores=NSUB
)

_call = functools.partial(
    pl.kernel,
    out_type=jax.ShapeDtypeStruct((NROWS, E), jnp.float32),
    mesh=_mesh,
    scratch_types=[
        pltpu.VMEM((A, NG, KSUB, GSUB), jnp.int32),
        pltpu.VMEM((2, G, E), jnp.float32),
        pltpu.SemaphoreType.DMA,
        pltpu.SemaphoreType.DMA,
    ],
)(_sc_body)


def kernel(position, table0, table1, table2):
    idx = position.reshape(NROWS, A).T.reshape(A, NW, NG, KSUB, GSUB)
    idx = idx.transpose(1, 0, 2, 3, 4)
    out = _call(table0, table1, table2, idx)
    return out.reshape(N, S, E)
